# baseline (speedup 1.0000x reference)
import functools

import jax
import jax.numpy as jnp
from jax import lax
from jax.experimental import pallas as pl
from jax.experimental.pallas import tpu as pltpu
from jax.experimental.pallas import tpu_sc as plsc

_LANES = 16
_DIM = 64
_CODES = 16

_mesh = plsc.VectorSubcoreMesh(core_axis_name="c", subcore_axis_name="s")


@functools.partial(
    pl.kernel,
    out_type=(
        jax.ShapeDtypeStruct((_CODES,), jnp.float32),
        jax.ShapeDtypeStruct((_DIM,), jnp.float32),
    ),
    mesh=_mesh,
    compiler_params=pltpu.CompilerParams(needs_layout_passes=False),
    scratch_types=[
        pltpu.VMEM((_DIM,), jnp.float32),
        pltpu.VMEM((_CODES * _DIM,), jnp.float32),
        pltpu.VMEM((_CODES,), jnp.float32),
        pltpu.VMEM((_DIM,), jnp.float32),
    ],
)
def _vq_kernel(x_hbm, cb_hbm, onehot_hbm, resid_hbm, x_v, cb_v, oh_v, r_v):
    @pl.when((lax.axis_index("c") == 0) & (lax.axis_index("s") == 0))
    def _():
        pltpu.sync_copy(x_hbm, x_v)
        pltpu.sync_copy(cb_hbm, cb_v)
        lanes = lax.iota(jnp.int32, _LANES)
        row_base = lanes * _DIM
        acc = jnp.zeros((_LANES,), jnp.float32)
        for d in range(_DIM):
            d_splat = jnp.full((_LANES,), d, jnp.int32)
            col = plsc.load_gather(cb_v, [row_base + d])
            xb = plsc.load_gather(x_v, [d_splat])
            t = xb - col
            acc = acc + t * t
        m = jnp.min(acc)
        idx = plsc.all_reduce_ffs(acc == m)
        oh_v[...] = jnp.where(lanes == idx, 1.0, 0.0).astype(jnp.float32)
        for i in range(_DIM // _LANES):
            xi = x_v[pl.ds(_LANES * i, _LANES)]
            row = plsc.load_gather(cb_v, [idx * _DIM + lanes + _LANES * i])
            r_v[pl.ds(_LANES * i, _LANES)] = xi - row
        pltpu.sync_copy(oh_v, onehot_hbm)
        pltpu.sync_copy(r_v, resid_hbm)


def kernel(inputs, codebook):
    x = jnp.reshape(inputs, (_DIM,))
    cb = jnp.reshape(codebook, (_CODES * _DIM,))
    oh, resid = _vq_kernel(x, cb)
    return jnp.reshape(oh, (1, _CODES)), jnp.reshape(resid, (1, 1, _DIM))


# speedup vs baseline: 1.1387x; 1.1387x over previous
import functools

import jax
import jax.numpy as jnp
from jax import lax
from jax.experimental import pallas as pl
from jax.experimental.pallas import tpu as pltpu
from jax.experimental.pallas import tpu_sc as plsc

_LANES = 16
_DIM = 64
_CODES = 16

_mesh = plsc.VectorSubcoreMesh(
    core_axis_name="c", subcore_axis_name="s", num_cores=1, num_subcores=16
)


@functools.partial(
    pl.kernel,
    out_type=(
        jax.ShapeDtypeStruct((_CODES,), jnp.float32),
        jax.ShapeDtypeStruct((_DIM,), jnp.float32),
    ),
    mesh=_mesh,
    compiler_params=pltpu.CompilerParams(needs_layout_passes=False),
    scratch_types=[
        pltpu.VMEM((_DIM,), jnp.float32),
        pltpu.VMEM((_CODES * _DIM,), jnp.float32),
        pltpu.VMEM((_CODES,), jnp.float32),
        pltpu.VMEM((_DIM,), jnp.float32),
        pltpu.SemaphoreType.DMA,
        pltpu.SemaphoreType.DMA,
    ],
)
def _vq_kernel(x_hbm, cb_hbm, onehot_hbm, resid_hbm, x_v, cb_v, oh_v, r_v,
               sem_a, sem_b):
    @pl.when((lax.axis_index("c") == 0) & (lax.axis_index("s") == 0))
    def _():
        in_a = pltpu.async_copy(x_hbm, x_v, sem_a)
        in_b = pltpu.async_copy(cb_hbm, cb_v, sem_b)
        in_a.wait()
        in_b.wait()
        lanes = lax.iota(jnp.int32, _LANES)
        row_base = lanes * _DIM
        acc0 = jnp.zeros((_LANES,), jnp.float32)
        acc1 = jnp.zeros((_LANES,), jnp.float32)
        for d in range(0, _DIM, 2):
            col0 = plsc.load_gather(cb_v, [row_base + d])
            xb0 = plsc.load_gather(x_v, [jnp.full((_LANES,), d, jnp.int32)])
            col1 = plsc.load_gather(cb_v, [row_base + d + 1])
            xb1 = plsc.load_gather(x_v, [jnp.full((_LANES,), d + 1, jnp.int32)])
            t0 = xb0 - col0
            t1 = xb1 - col1
            acc0 = acc0 + t0 * t0
            acc1 = acc1 + t1 * t1
        acc = acc0 + acc1
        m = jnp.min(acc)
        idx = plsc.all_reduce_ffs(acc == m)
        oh_v[...] = jnp.where(lanes == idx, 1.0, 0.0).astype(jnp.float32)
        for i in range(_DIM // _LANES):
            xi = x_v[pl.ds(_LANES * i, _LANES)]
            row = plsc.load_gather(cb_v, [idx * _DIM + lanes + _LANES * i])
            r_v[pl.ds(_LANES * i, _LANES)] = xi - row
        out_a = pltpu.async_copy(oh_v, onehot_hbm, sem_a)
        out_b = pltpu.async_copy(r_v, resid_hbm, sem_b)
        out_a.wait()
        out_b.wait()


def kernel(inputs, codebook):
    x = jnp.reshape(inputs, (_DIM,))
    cb = jnp.reshape(codebook, (_CODES * _DIM,))
    oh, resid = _vq_kernel(x, cb)
    return jnp.reshape(oh, (1, _CODES)), jnp.reshape(resid, (1, 1, _DIM))
